# baseline (speedup 1.0000x reference)
import functools

import jax
import jax.numpy as jnp
import numpy as np
from jax import lax
from jax.experimental import pallas as pl
from jax.experimental.pallas import tpu as pltpu
from jax.experimental.pallas import tpu_sc as plsc

N_V = 10000
N_E = 5000
N_PAIRS = 160000
C = 256
H = 4
D_IN = C // 2
D_HEAD = D_IN // H
TDW = 160
TDW32 = TDW // 2
SEGS = TDW32 // 16

NC = 2
NS = 16
P_PAD = 163840
SPLIT_A = (57, 23)
SPLIT_B = (228, 92)

NE_ACC = 5120
NV_ACC = 10016

_CP = np.array([32 * (f // 32) + 2 * (f % 16) + (f % 32) // 16
                for f in range(TDW)], dtype=np.int32)


def _seg_sum_body(nacc, ch, n0, n1, data, gidx, sidx, zeros, out, acc, gi_v,
                  si_v, bf_v, rows_v, sem):
    c = lax.axis_index("c")
    s = lax.axis_index("s")
    rpt = nacc // NS

    pltpu.sync_copy(zeros.at[pl.ds(0, rpt)], acc.at[pl.ds(s * rpt, rpt)])
    plsc.subcore_barrier()

    def run(base, n):
        pltpu.sync_copy(gidx.at[pl.ds(base, n)], gi_v.at[pl.ds(0, n)])
        pltpu.sync_copy(sidx.at[pl.ds(base, n)], si_v.at[pl.ds(0, n)])

        pltpu.async_copy(data.at[gi_v.at[0]], bf_v.at[0], sem)

        def chunk(j, carry):
            p = lax.rem(j, 2)
            pltpu.make_async_copy(data.at[gi_v.at[j]], bf_v.at[p],
                                  sem).wait()

            @pl.when(j + 1 < n)
            def _():
                pltpu.async_copy(data.at[gi_v.at[j + 1]], bf_v.at[1 - p],
                                 sem)

            def widen(r, carry2):
                for g in range(SEGS):
                    v = bf_v[p, r, pl.ds(g * 16, 16)]
                    lo = plsc.bitcast(lax.shift_left(v, 16), jnp.float32)
                    hi = plsc.bitcast(
                        jnp.bitwise_and(v, jnp.int32(-65536)), jnp.float32)
                    rows_v[p, r, pl.ds(g * 32, 16)] = lo
                    rows_v[p, r, pl.ds(g * 32 + 16, 16)] = hi
                return carry2

            lax.fori_loop(0, ch, widen, 0)
            pltpu.sync_copy(rows_v.at[p], acc.at[si_v.at[j]], add=True)
            return carry

        lax.fori_loop(0, n, chunk, 0)

    @pl.when(c == 0)
    def _():
        run(s * n0, n0)

    @pl.when(c == 1)
    def _():
        run(NS * n0 + s * n1, n1)

    plsc.subcore_barrier()

    r0 = s * rpt
    pltpu.sync_copy(acc.at[pl.ds(r0, rpt)],
                    out.at[pl.ds(c * nacc + r0, rpt)])


def _seg_sum(data, gidx, sidx, zeros, nacc, split):
    n0, n1 = split
    ch = P_PAD // ((n0 + n1) * NS)
    body = functools.partial(_seg_sum_body, nacc, ch, n0, n1)
    f = pl.kernel(
        body,
        out_type=jax.ShapeDtypeStruct((NC * nacc, TDW), jnp.float32),
        mesh=plsc.VectorSubcoreMesh(core_axis_name="c", subcore_axis_name="s"),
        scratch_types=[
            pltpu.VMEM_SHARED((nacc, TDW), jnp.float32),
            pltpu.VMEM((max(n0, n1), ch), jnp.int32),
            pltpu.VMEM((max(n0, n1), ch), jnp.int32),
            pltpu.VMEM((2, ch, TDW32), jnp.int32),
            pltpu.VMEM((2, ch, TDW), jnp.float32),
            pltpu.SemaphoreType.DMA,
        ],
        compiler_params=pltpu.CompilerParams(use_tc_tiling_on_sc=False,
                                             needs_layout_passes=False),
    )
    return f(data, gidx.reshape(-1, ch), sidx.reshape(-1, ch), zeros)


def _pack_i32(t_bf16):
    n = t_bf16.shape[0]
    return lax.bitcast_convert_type(
        t_bf16.reshape(n, TDW32, 2), jnp.int32)


def _theta_body(x_ref, w_ref, b_ref, o_ref):
    xt = jnp.dot(x_ref[...], w_ref[...],
                 preferred_element_type=jnp.float32) + b_ref[...]
    extra = jnp.broadcast_to(
        (lax.broadcasted_iota(jnp.int32, (1, 32), 1) == 0)
        .astype(jnp.float32), (xt.shape[0], 32))
    o_ref[...] = jnp.concatenate([xt, extra], axis=1).astype(jnp.bfloat16)


def _theta(x, wcat, bcat):
    n = x.shape[0]
    blk = 2000
    return pl.pallas_call(
        _theta_body,
        grid=(n // blk,),
        in_specs=[
            pl.BlockSpec((blk, D_IN), lambda i: (i, 0)),
            pl.BlockSpec((D_IN, D_IN), lambda i: (0, 0)),
            pl.BlockSpec((1, D_IN), lambda i: (0, 0)),
        ],
        out_specs=pl.BlockSpec((blk, TDW), lambda i: (i, 0)),
        out_shape=jax.ShapeDtypeStruct((n, TDW), jnp.bfloat16),
    )(x, wcat, bcat.reshape(1, D_IN))


def _edge_stage_body(agg_ref, ae_ref, bd_ref, sel_ref, z_ref):
    s = agg_ref[0] + agg_ref[1]
    deg = jnp.maximum(s[:, D_IN:D_IN + 1], 1.0)
    y = s[:, :D_IN] / deg
    p = y * ae_ref[...]
    alpha = jnp.dot(p, bd_ref[...],
                    preferred_element_type=jnp.float32)
    alpha = jnp.where(alpha > 0, alpha, 0.2 * alpha)
    m = jnp.max(alpha, axis=0, keepdims=True)
    e = jnp.exp(alpha - m)
    extra = jnp.dot(e, sel_ref[...],
                    preferred_element_type=jnp.float32)
    z_ref[...] = jnp.concatenate([y * e, extra],
                                 axis=1).astype(jnp.bfloat16)


def _edge_stage(agg2, aeperm, bd, sel):
    return pl.pallas_call(
        _edge_stage_body,
        in_specs=[
            pl.BlockSpec((2, N_E, TDW), lambda: (0, 0, 0)),
            pl.BlockSpec((1, D_IN), lambda: (0, 0)),
            pl.BlockSpec((D_IN, D_IN), lambda: (0, 0)),
            pl.BlockSpec((D_IN, 32), lambda: (0, 0)),
        ],
        out_specs=pl.BlockSpec((N_E, TDW), lambda: (0, 0)),
        out_shape=jax.ShapeDtypeStruct((N_E, TDW), jnp.bfloat16),
    )(agg2, aeperm.reshape(1, D_IN), bd, sel)


def _vertex_out(s, exp_ref, m2_ref, xres_ref):
    den = jnp.dot(s[:, D_IN:], exp_ref[...],
                  preferred_element_type=jnp.float32)
    out = s[:, :D_IN] / jnp.maximum(den, 1e-12)
    out = jnp.where(out > 0, out, jnp.exp(out) - 1.0)
    return xres_ref[...] + jnp.dot(out, m2_ref[...],
                                   preferred_element_type=jnp.float32)


def _vertex_theta_body(agg_ref, xres_ref, exp_ref, m2_ref, w_ref, b_ref,
                       y_ref, xt_ref):
    y = _vertex_out(agg_ref[0] + agg_ref[1], exp_ref, m2_ref, xres_ref)
    y_ref[...] = y
    xt = jnp.dot(y, w_ref[...],
                 preferred_element_type=jnp.float32) + b_ref[...]
    extra = jnp.broadcast_to(
        (lax.broadcasted_iota(jnp.int32, (1, 32), 1) == 0)
        .astype(jnp.float32), (xt.shape[0], 32))
    xt_ref[...] = jnp.concatenate([xt, extra], axis=1).astype(jnp.bfloat16)


def _vertex_theta(agg2, xres, expand, m2, wcat, bcat):
    blk = 2000
    return pl.pallas_call(
        _vertex_theta_body,
        grid=(N_V // blk,),
        in_specs=[
            pl.BlockSpec((2, blk, TDW), lambda i: (0, i, 0)),
            pl.BlockSpec((blk, D_IN), lambda i: (i, 0)),
            pl.BlockSpec((32, D_IN), lambda i: (0, 0)),
            pl.BlockSpec((D_IN, D_IN), lambda i: (0, 0)),
            pl.BlockSpec((D_IN, D_IN), lambda i: (0, 0)),
            pl.BlockSpec((1, D_IN), lambda i: (0, 0)),
        ],
        out_specs=[
            pl.BlockSpec((blk, D_IN), lambda i: (i, 0)),
            pl.BlockSpec((blk, TDW), lambda i: (i, 0)),
        ],
        out_shape=[
            jax.ShapeDtypeStruct((N_V, D_IN), jnp.float32),
            jax.ShapeDtypeStruct((N_V, TDW), jnp.bfloat16),
        ],
    )(agg2, xres, expand, m2, wcat, bcat.reshape(1, D_IN))


def _vertex_cat_body(agg_ref, xres_ref, exp_ref, m2_ref, y1_ref, o_ref):
    y2 = _vertex_out(agg_ref[0] + agg_ref[1], exp_ref, m2_ref, xres_ref)
    o_ref[...] = jnp.concatenate([y1_ref[...], y2], axis=1)


def _vertex_cat(agg2, xres, expand, m2, y1):
    blk = 2000
    return pl.pallas_call(
        _vertex_cat_body,
        grid=(N_V // blk,),
        in_specs=[
            pl.BlockSpec((2, blk, TDW), lambda i: (0, i, 0)),
            pl.BlockSpec((blk, D_IN), lambda i: (i, 0)),
            pl.BlockSpec((32, D_IN), lambda i: (0, 0)),
            pl.BlockSpec((D_IN, D_IN), lambda i: (0, 0)),
            pl.BlockSpec((blk, D_IN), lambda i: (i, 0)),
        ],
        out_specs=pl.BlockSpec((blk, C), lambda i: (i, 0)),
        out_shape=jax.ShapeDtypeStruct((N_V, C), jnp.float32),
    )(agg2, xres, expand, m2, y1)


def _gather_scatter(xt_bf, aeperm, consts):
    bd, sel, zeros, vg, es, eg, vs = consts
    agg_e = _seg_sum(_pack_i32(xt_bf), vg, es, zeros, NE_ACC, SPLIT_A)
    agg_e = agg_e.reshape(NC, NE_ACC, TDW)[:, :N_E, :]
    z = _edge_stage(agg_e, aeperm, bd, sel)
    agg_v = _seg_sum(_pack_i32(z), eg, vs, zeros, NV_ACC, SPLIT_B)
    return agg_v.reshape(NC, NV_ACC, TDW)[:, :N_V, :]


def kernel(x, v_idx, e_idx, W1, b1, ae1, W2, b2, ae2):
    f32 = jnp.float32
    x1, x2 = x[:, :D_IN], x[:, D_IN:]
    w1c = jnp.transpose(W1, (1, 0, 2)).reshape(D_IN, D_IN)
    w2c = jnp.transpose(W2, (1, 0, 2)).reshape(D_IN, D_IN)
    b1c, b2c = b1.reshape(D_IN), b2.reshape(D_IN)
    ae1p = ae1.reshape(D_IN)[_CP[:D_IN]]
    ae2p = ae2.reshape(D_IN)[_CP[:D_IN]]

    heads = jnp.arange(D_IN, dtype=jnp.int32) // D_HEAD
    bd = (heads[:, None] == heads[None, :]).astype(f32)
    col32 = np.arange(32)
    sel = np.zeros((D_IN, 32), np.float32)
    for j in range(H):
        sel[32 * j, j] = 1.0
    sel = jnp.asarray(sel)
    ij = [int(np.where(_CP[128:160] == 128 + j)[0][0]) for j in range(H)]
    expand = np.zeros((32, D_IN), np.float32)
    for j in range(H):
        expand[ij[j], 32 * j:32 * (j + 1)] = 1.0
    expand = jnp.asarray(expand)
    cp2 = _CP[_CP[:D_IN]]
    m2 = np.zeros((D_IN, D_IN), np.float32)
    for f2 in range(D_IN):
        m2[f2, cp2[f2]] = 1.0
    m2 = jnp.asarray(m2)

    zeros = jnp.zeros((NV_ACC // NS, TDW), f32)
    npad = P_PAD - N_PAIRS
    v32 = v_idx.astype(jnp.int32)
    e32 = e_idx.astype(jnp.int32)
    pad0 = jnp.zeros((npad,), jnp.int32)
    vg = jnp.concatenate([v32, pad0])
    eg = jnp.concatenate([e32, pad0])
    es = jnp.concatenate([e32, jnp.full((npad,), N_E, jnp.int32)])
    vs = jnp.concatenate([v32, jnp.full((npad,), N_V, jnp.int32)])

    consts = (bd, sel, zeros, vg, es, eg, vs)
    xt1 = _theta(x2, w1c, b1c)
    agg_v1 = _gather_scatter(xt1, ae1p, consts)
    y1, xt2 = _vertex_theta(agg_v1, x1, expand, m2, w2c, b2c)
    agg_v2 = _gather_scatter(xt2, ae2p, consts)
    return _vertex_cat(agg_v2, x2, expand, m2, y1)


# speedup vs baseline: 1.2218x; 1.2218x over previous
import functools

import jax
import jax.numpy as jnp
from jax import lax
from jax.experimental import pallas as pl
from jax.experimental.pallas import tpu as pltpu
from jax.experimental.pallas import tpu_sc as plsc

N_V = 10000
N_E = 5000
N_PAIRS = 160000
C = 256
H = 4
D_IN = C // 2
D_HEAD = D_IN // H
WID = 144

NC = 2
NS = 16
P_PAD = 163840
SPLIT_A = (57, 23)
SPLIT_B = (114, 46)

NE_ACC = 5120
NV_ACC = 10016


def _seg_sum_body(nacc, ch, n0, n1, deep, data, gidx, sidx, zeros, out, acc,
                  gi_v, si_v, rows_v, sem, sem_s):
    c = lax.axis_index("c")
    s = lax.axis_index("s")
    rpt = nacc // NS

    pltpu.sync_copy(zeros.at[pl.ds(0, rpt)], acc.at[pl.ds(s * rpt, rpt)])
    plsc.subcore_barrier()

    def run(base, n):
        pltpu.sync_copy(gidx.at[pl.ds(base, n)], gi_v.at[pl.ds(0, n)])
        pltpu.sync_copy(sidx.at[pl.ds(base, n)], si_v.at[pl.ds(0, n)])

        pltpu.async_copy(data.at[gi_v.at[0]], rows_v.at[0], sem)

        if deep:
            def chunk(j, carry):
                p = lax.rem(j, 3)
                pltpu.make_async_copy(data.at[gi_v.at[j]], rows_v.at[p],
                                      sem).wait()

                @pl.when(j >= 2)
                def _():
                    q = lax.rem(j - 2, 3)
                    pltpu.make_async_copy(rows_v.at[q],
                                          acc.at[si_v.at[j - 2]],
                                          sem_s).wait()

                @pl.when(j + 1 < n)
                def _():
                    pltpu.async_copy(data.at[gi_v.at[j + 1]],
                                     rows_v.at[lax.rem(j + 1, 3)], sem)

                pltpu.async_copy(rows_v.at[p], acc.at[si_v.at[j]], sem_s,
                                 add=True)
                return carry

            lax.fori_loop(0, n, chunk, 0)
            pltpu.make_async_copy(rows_v.at[lax.rem(n - 2, 3)],
                                  acc.at[si_v.at[n - 2]], sem_s).wait()
            pltpu.make_async_copy(rows_v.at[lax.rem(n - 1, 3)],
                                  acc.at[si_v.at[n - 1]], sem_s).wait()
        else:
            def chunk(j, carry):
                p = lax.rem(j, 2)
                pltpu.make_async_copy(data.at[gi_v.at[j]], rows_v.at[p],
                                      sem).wait()

                @pl.when(j + 1 < n)
                def _():
                    pltpu.async_copy(data.at[gi_v.at[j + 1]],
                                     rows_v.at[1 - p], sem)

                pltpu.sync_copy(rows_v.at[p], acc.at[si_v.at[j]], add=True)
                return carry

            lax.fori_loop(0, n, chunk, 0)

    @pl.when(c == 0)
    def _():
        run(s * n0, n0)

    @pl.when(c == 1)
    def _():
        run(NS * n0 + s * n1, n1)

    plsc.subcore_barrier()

    r0 = s * rpt
    pltpu.sync_copy(acc.at[pl.ds(r0, rpt)],
                    out.at[pl.ds(c * nacc + r0, rpt)])


def _seg_sum(data, gidx, sidx, zeros, nacc, split, deep):
    n0, n1 = split
    ch = P_PAD // ((n0 + n1) * NS)
    body = functools.partial(_seg_sum_body, nacc, ch, n0, n1, deep)
    f = pl.kernel(
        body,
        out_type=jax.ShapeDtypeStruct((NC * nacc, WID), jnp.float32),
        mesh=plsc.VectorSubcoreMesh(core_axis_name="c", subcore_axis_name="s"),
        scratch_types=[
            pltpu.VMEM_SHARED((nacc, WID), jnp.float32),
            pltpu.VMEM((max(n0, n1), ch), jnp.int32),
            pltpu.VMEM((max(n0, n1), ch), jnp.int32),
            pltpu.VMEM((3 if deep else 2, ch, WID), jnp.float32),
            pltpu.SemaphoreType.DMA,
            pltpu.SemaphoreType.DMA,
        ],
        compiler_params=pltpu.CompilerParams(use_tc_tiling_on_sc=False),
    )
    return f(data, gidx.reshape(-1, ch), sidx.reshape(-1, ch), zeros)


def _theta_body(x_ref, w_ref, b_ref, o_ref):
    xt = jnp.dot(x_ref[...], w_ref[...],
                 preferred_element_type=jnp.float32) + b_ref[...]
    extra = jnp.broadcast_to(
        (lax.broadcasted_iota(jnp.int32, (1, 16), 1) == 0)
        .astype(jnp.float32), (xt.shape[0], 16))
    o_ref[...] = jnp.concatenate([xt, extra], axis=1)


def _theta(x, wcat, bcat):
    n = x.shape[0]
    blk = 2000
    return pl.pallas_call(
        _theta_body,
        grid=(n // blk,),
        in_specs=[
            pl.BlockSpec((blk, D_IN), lambda i: (i, 0)),
            pl.BlockSpec((D_IN, D_IN), lambda i: (0, 0)),
            pl.BlockSpec((1, D_IN), lambda i: (0, 0)),
        ],
        out_specs=pl.BlockSpec((blk, WID), lambda i: (i, 0)),
        out_shape=jax.ShapeDtypeStruct((n, WID), jnp.float32),
    )(x, wcat, bcat.reshape(1, D_IN))


def _edge_stage_body(agg_ref, ae_ref, bd_ref, sel_ref, z_ref):
    s = agg_ref[0] + agg_ref[1]
    deg = jnp.maximum(s[:, D_IN:D_IN + 1], 1.0)
    y = s[:, :D_IN] / deg
    p = y * ae_ref[...]
    alpha = jnp.dot(p, bd_ref[...],
                    preferred_element_type=jnp.float32)
    alpha = jnp.where(alpha > 0, alpha, 0.2 * alpha)
    m = jnp.max(alpha, axis=0, keepdims=True)
    e = jnp.exp(alpha - m)
    extra = jnp.dot(e, sel_ref[...],
                    preferred_element_type=jnp.float32)
    z_ref[...] = jnp.concatenate([y * e, extra], axis=1)


def _edge_stage(agg2, aecat, bd, sel):
    return pl.pallas_call(
        _edge_stage_body,
        in_specs=[
            pl.BlockSpec((2, N_E, WID), lambda: (0, 0, 0)),
            pl.BlockSpec((1, D_IN), lambda: (0, 0)),
            pl.BlockSpec((D_IN, D_IN), lambda: (0, 0)),
            pl.BlockSpec((D_IN, 16), lambda: (0, 0)),
        ],
        out_specs=pl.BlockSpec((N_E, WID), lambda: (0, 0)),
        out_shape=jax.ShapeDtypeStruct((N_E, WID), jnp.float32),
    )(agg2, aecat.reshape(1, D_IN), bd, sel)


def _vertex_theta_body(agg_ref, xres_ref, exp_ref, w_ref, b_ref, y_ref,
                       xt_ref):
    s = agg_ref[0] + agg_ref[1]
    den = jnp.dot(s[:, D_IN:], exp_ref[...],
                  preferred_element_type=jnp.float32)
    out = s[:, :D_IN] / jnp.maximum(den, 1e-12)
    out = jnp.where(out > 0, out, jnp.exp(out) - 1.0)
    y = xres_ref[...] + out
    y_ref[...] = y
    xt = jnp.dot(y, w_ref[...],
                 preferred_element_type=jnp.float32) + b_ref[...]
    extra = jnp.broadcast_to(
        (lax.broadcasted_iota(jnp.int32, (1, 16), 1) == 0)
        .astype(jnp.float32), (xt.shape[0], 16))
    xt_ref[...] = jnp.concatenate([xt, extra], axis=1)


def _vertex_theta(agg2, xres, expand, wcat, bcat):
    blk = 2000
    return pl.pallas_call(
        _vertex_theta_body,
        grid=(N_V // blk,),
        in_specs=[
            pl.BlockSpec((2, blk, WID), lambda i: (0, i, 0)),
            pl.BlockSpec((blk, D_IN), lambda i: (i, 0)),
            pl.BlockSpec((16, D_IN), lambda i: (0, 0)),
            pl.BlockSpec((D_IN, D_IN), lambda i: (0, 0)),
            pl.BlockSpec((1, D_IN), lambda i: (0, 0)),
        ],
        out_specs=[
            pl.BlockSpec((blk, D_IN), lambda i: (i, 0)),
            pl.BlockSpec((blk, WID), lambda i: (i, 0)),
        ],
        out_shape=[
            jax.ShapeDtypeStruct((N_V, D_IN), jnp.float32),
            jax.ShapeDtypeStruct((N_V, WID), jnp.float32),
        ],
    )(agg2, xres, expand, wcat, bcat.reshape(1, D_IN))


def _vertex_cat_body(agg_ref, xres_ref, exp_ref, y1_ref, o_ref):
    s = agg_ref[0] + agg_ref[1]
    den = jnp.dot(s[:, D_IN:], exp_ref[...],
                  preferred_element_type=jnp.float32)
    out = s[:, :D_IN] / jnp.maximum(den, 1e-12)
    out = jnp.where(out > 0, out, jnp.exp(out) - 1.0)
    o_ref[...] = jnp.concatenate([y1_ref[...], xres_ref[...] + out], axis=1)


def _vertex_cat(agg2, xres, expand, y1):
    blk = 2000
    return pl.pallas_call(
        _vertex_cat_body,
        grid=(N_V // blk,),
        in_specs=[
            pl.BlockSpec((2, blk, WID), lambda i: (0, i, 0)),
            pl.BlockSpec((blk, D_IN), lambda i: (i, 0)),
            pl.BlockSpec((16, D_IN), lambda i: (0, 0)),
            pl.BlockSpec((blk, D_IN), lambda i: (i, 0)),
        ],
        out_specs=pl.BlockSpec((blk, C), lambda i: (i, 0)),
        out_shape=jax.ShapeDtypeStruct((N_V, C), jnp.float32),
    )(agg2, xres, expand, y1)


def _vertex_stage_body(agg_ref, xres_ref, exp_ref, y_ref):
    s = agg_ref[0] + agg_ref[1]
    den = jnp.dot(s[:, D_IN:], exp_ref[...],
                  preferred_element_type=jnp.float32)
    out = s[:, :D_IN] / jnp.maximum(den, 1e-12)
    out = jnp.where(out > 0, out, jnp.exp(out) - 1.0)
    y_ref[...] = xres_ref[...] + out


def _vertex_stage(agg2, xres, expand):
    blk = 2000
    return pl.pallas_call(
        _vertex_stage_body,
        grid=(N_V // blk,),
        in_specs=[
            pl.BlockSpec((2, blk, WID), lambda i: (0, i, 0)),
            pl.BlockSpec((blk, D_IN), lambda i: (i, 0)),
            pl.BlockSpec((16, D_IN), lambda i: (0, 0)),
        ],
        out_specs=pl.BlockSpec((blk, D_IN), lambda i: (i, 0)),
        out_shape=jax.ShapeDtypeStruct((N_V, D_IN), jnp.float32),
    )(agg2, xres, expand)


def _gather_scatter(xt, aecat, consts):
    bd, sel, expand, zeros, vg, es, eg, vs = consts
    agg_e = _seg_sum(xt, vg, es, zeros, NE_ACC, SPLIT_A, True)
    agg_e = agg_e.reshape(NC, NE_ACC, WID)[:, :N_E, :]
    z = _edge_stage(agg_e, aecat, bd, sel)
    agg_v = _seg_sum(z, eg, vs, zeros, NV_ACC, SPLIT_B, False)
    return agg_v.reshape(NC, NV_ACC, WID)[:, :N_V, :]


def kernel(x, v_idx, e_idx, W1, b1, ae1, W2, b2, ae2):
    f32 = jnp.float32
    x1, x2 = x[:, :D_IN], x[:, D_IN:]
    w1c = jnp.transpose(W1, (1, 0, 2)).reshape(D_IN, D_IN)
    w2c = jnp.transpose(W2, (1, 0, 2)).reshape(D_IN, D_IN)
    b1c, b2c = b1.reshape(D_IN), b2.reshape(D_IN)
    ae1c, ae2c = ae1.reshape(D_IN), ae2.reshape(D_IN)

    heads = jnp.arange(D_IN, dtype=jnp.int32) // D_HEAD
    bd = (heads[:, None] == heads[None, :]).astype(f32)
    col = jnp.arange(16, dtype=jnp.int32)
    sel = ((jnp.arange(D_IN)[:, None] == col[None, :] * D_HEAD)
           & (col[None, :] < H)).astype(f32)
    expand = (col[:, None] == heads[None, :]).astype(f32)

    zeros = jnp.zeros((NV_ACC // NS, WID), f32)
    npad = P_PAD - N_PAIRS
    v32 = v_idx.astype(jnp.int32)
    e32 = e_idx.astype(jnp.int32)
    pad0 = jnp.zeros((npad,), jnp.int32)
    vg = jnp.concatenate([v32, pad0])
    eg = jnp.concatenate([e32, pad0])
    es = jnp.concatenate([e32, jnp.full((npad,), N_E, jnp.int32)])
    vs = jnp.concatenate([v32, jnp.full((npad,), N_V, jnp.int32)])

    consts = (bd, sel, expand, zeros, vg, es, eg, vs)
    xt1 = _theta(x2, w1c, b1c)
    agg_v1 = _gather_scatter(xt1, ae1c, consts)
    y1, xt2 = _vertex_theta(agg_v1, x1, expand, w2c, b2c)
    agg_v2 = _gather_scatter(xt2, ae2c, consts)
    return _vertex_cat(agg_v2, x2, expand, y1)


# speedup vs baseline: 1.2862x; 1.0527x over previous
import functools

import jax
import jax.numpy as jnp
from jax import lax
from jax.experimental import pallas as pl
from jax.experimental.pallas import tpu as pltpu
from jax.experimental.pallas import tpu_sc as plsc

N_V = 10000
N_E = 5000
N_PAIRS = 160000
C = 256
H = 4
D_IN = C // 2
D_HEAD = D_IN // H
WID = 144

NC = 2
NS = 16
P_PAD = 163840
SPLIT_A = (60, 20)
SPLIT_B = (118, 42)

NE_ACC = 5120
NV_ACC = 10016


def _seg_sum_body(nacc, ch, n0, n1, data, gidx, sidx, zeros, out, acc, gi_v,
                  si_v, rows_v, sem):
    c = lax.axis_index("c")
    s = lax.axis_index("s")
    rpt = nacc // NS

    pltpu.sync_copy(zeros.at[pl.ds(0, rpt)], acc.at[pl.ds(s * rpt, rpt)])
    plsc.subcore_barrier()

    def run(base, n):
        pltpu.sync_copy(gidx.at[pl.ds(base, n)], gi_v.at[pl.ds(0, n)])
        pltpu.sync_copy(sidx.at[pl.ds(base, n)], si_v.at[pl.ds(0, n)])

        pltpu.async_copy(data.at[gi_v.at[0]], rows_v.at[0], sem)

        def chunk(j, carry):
            p = lax.rem(j, 2)
            pltpu.make_async_copy(data.at[gi_v.at[j]], rows_v.at[p],
                                  sem).wait()

            @pl.when(j + 1 < n)
            def _():
                pltpu.async_copy(data.at[gi_v.at[j + 1]], rows_v.at[1 - p],
                                 sem)

            pltpu.sync_copy(rows_v.at[p], acc.at[si_v.at[j]], add=True)
            return carry

        lax.fori_loop(0, n, chunk, 0)

    @pl.when(c == 0)
    def _():
        run(s * n0, n0)

    @pl.when(c == 1)
    def _():
        run(NS * n0 + s * n1, n1)

    plsc.subcore_barrier()

    r0 = s * rpt
    pltpu.sync_copy(acc.at[pl.ds(r0, rpt)],
                    out.at[pl.ds(c * nacc + r0, rpt)])


def _seg_sum(data, gidx, sidx, zeros, nacc, split):
    n0, n1 = split
    ch = P_PAD // ((n0 + n1) * NS)
    body = functools.partial(_seg_sum_body, nacc, ch, n0, n1)
    f = pl.kernel(
        body,
        out_type=jax.ShapeDtypeStruct((NC * nacc, WID), jnp.float32),
        mesh=plsc.VectorSubcoreMesh(core_axis_name="c", subcore_axis_name="s"),
        scratch_types=[
            pltpu.VMEM_SHARED((nacc, WID), jnp.float32),
            pltpu.VMEM((max(n0, n1), ch), jnp.int32),
            pltpu.VMEM((max(n0, n1), ch), jnp.int32),
            pltpu.VMEM((2, ch, WID), jnp.float32),
            pltpu.SemaphoreType.DMA,
        ],
        compiler_params=pltpu.CompilerParams(use_tc_tiling_on_sc=False),
    )
    return f(data, gidx.reshape(-1, ch), sidx.reshape(-1, ch), zeros)


def _theta_body(x_ref, w_ref, b_ref, o_ref):
    xt = jnp.dot(x_ref[...], w_ref[...],
                 preferred_element_type=jnp.float32) + b_ref[...]
    extra = jnp.broadcast_to(
        (lax.broadcasted_iota(jnp.int32, (1, 16), 1) == 0)
        .astype(jnp.float32), (xt.shape[0], 16))
    o_ref[...] = jnp.concatenate([xt, extra], axis=1)


def _theta(x, wcat, bcat):
    n = x.shape[0]
    blk = 2000
    return pl.pallas_call(
        _theta_body,
        grid=(n // blk,),
        in_specs=[
            pl.BlockSpec((blk, D_IN), lambda i: (i, 0)),
            pl.BlockSpec((D_IN, D_IN), lambda i: (0, 0)),
            pl.BlockSpec((1, D_IN), lambda i: (0, 0)),
        ],
        out_specs=pl.BlockSpec((blk, WID), lambda i: (i, 0)),
        out_shape=jax.ShapeDtypeStruct((n, WID), jnp.float32),
    )(x, wcat, bcat.reshape(1, D_IN))


def _edge_stage_body(agg_ref, ae_ref, bd_ref, sel_ref, z_ref):
    s = agg_ref[0] + agg_ref[1]
    deg = jnp.maximum(s[:, D_IN:D_IN + 1], 1.0)
    y = s[:, :D_IN] / deg
    p = y * ae_ref[...]
    alpha = jnp.dot(p, bd_ref[...],
                    preferred_element_type=jnp.float32)
    alpha = jnp.where(alpha > 0, alpha, 0.2 * alpha)
    m = jnp.max(alpha, axis=0, keepdims=True)
    e = jnp.exp(alpha - m)
    extra = jnp.dot(e, sel_ref[...],
                    preferred_element_type=jnp.float32)
    z_ref[...] = jnp.concatenate([y * e, extra], axis=1)


def _edge_stage(agg2, aecat, bd, sel):
    return pl.pallas_call(
        _edge_stage_body,
        in_specs=[
            pl.BlockSpec((2, N_E, WID), lambda: (0, 0, 0)),
            pl.BlockSpec((1, D_IN), lambda: (0, 0)),
            pl.BlockSpec((D_IN, D_IN), lambda: (0, 0)),
            pl.BlockSpec((D_IN, 16), lambda: (0, 0)),
        ],
        out_specs=pl.BlockSpec((N_E, WID), lambda: (0, 0)),
        out_shape=jax.ShapeDtypeStruct((N_E, WID), jnp.float32),
    )(agg2, aecat.reshape(1, D_IN), bd, sel)


def _vertex_theta_body(agg_ref, xres_ref, exp_ref, w_ref, b_ref, y_ref,
                       xt_ref):
    s = agg_ref[0] + agg_ref[1]
    den = jnp.dot(s[:, D_IN:], exp_ref[...],
                  preferred_element_type=jnp.float32)
    out = s[:, :D_IN] / jnp.maximum(den, 1e-12)
    out = jnp.where(out > 0, out, jnp.exp(out) - 1.0)
    y = xres_ref[...] + out
    y_ref[...] = y
    xt = jnp.dot(y, w_ref[...],
                 preferred_element_type=jnp.float32) + b_ref[...]
    extra = jnp.broadcast_to(
        (lax.broadcasted_iota(jnp.int32, (1, 16), 1) == 0)
        .astype(jnp.float32), (xt.shape[0], 16))
    xt_ref[...] = jnp.concatenate([xt, extra], axis=1)


def _vertex_theta(agg2, xres, expand, wcat, bcat):
    blk = 2000
    return pl.pallas_call(
        _vertex_theta_body,
        grid=(N_V // blk,),
        in_specs=[
            pl.BlockSpec((2, blk, WID), lambda i: (0, i, 0)),
            pl.BlockSpec((blk, D_IN), lambda i: (i, 0)),
            pl.BlockSpec((16, D_IN), lambda i: (0, 0)),
            pl.BlockSpec((D_IN, D_IN), lambda i: (0, 0)),
            pl.BlockSpec((1, D_IN), lambda i: (0, 0)),
        ],
        out_specs=[
            pl.BlockSpec((blk, D_IN), lambda i: (i, 0)),
            pl.BlockSpec((blk, WID), lambda i: (i, 0)),
        ],
        out_shape=[
            jax.ShapeDtypeStruct((N_V, D_IN), jnp.float32),
            jax.ShapeDtypeStruct((N_V, WID), jnp.float32),
        ],
    )(agg2, xres, expand, wcat, bcat.reshape(1, D_IN))


def _vertex_cat_body(agg_ref, xres_ref, exp_ref, y1_ref, o_ref):
    s = agg_ref[0] + agg_ref[1]
    den = jnp.dot(s[:, D_IN:], exp_ref[...],
                  preferred_element_type=jnp.float32)
    out = s[:, :D_IN] / jnp.maximum(den, 1e-12)
    out = jnp.where(out > 0, out, jnp.exp(out) - 1.0)
    o_ref[...] = jnp.concatenate([y1_ref[...], xres_ref[...] + out], axis=1)


def _vertex_cat(agg2, xres, expand, y1):
    blk = 2000
    return pl.pallas_call(
        _vertex_cat_body,
        grid=(N_V // blk,),
        in_specs=[
            pl.BlockSpec((2, blk, WID), lambda i: (0, i, 0)),
            pl.BlockSpec((blk, D_IN), lambda i: (i, 0)),
            pl.BlockSpec((16, D_IN), lambda i: (0, 0)),
            pl.BlockSpec((blk, D_IN), lambda i: (i, 0)),
        ],
        out_specs=pl.BlockSpec((blk, C), lambda i: (i, 0)),
        out_shape=jax.ShapeDtypeStruct((N_V, C), jnp.float32),
    )(agg2, xres, expand, y1)


def _vertex_stage_body(agg_ref, xres_ref, exp_ref, y_ref):
    s = agg_ref[0] + agg_ref[1]
    den = jnp.dot(s[:, D_IN:], exp_ref[...],
                  preferred_element_type=jnp.float32)
    out = s[:, :D_IN] / jnp.maximum(den, 1e-12)
    out = jnp.where(out > 0, out, jnp.exp(out) - 1.0)
    y_ref[...] = xres_ref[...] + out


def _vertex_stage(agg2, xres, expand):
    blk = 2000
    return pl.pallas_call(
        _vertex_stage_body,
        grid=(N_V // blk,),
        in_specs=[
            pl.BlockSpec((2, blk, WID), lambda i: (0, i, 0)),
            pl.BlockSpec((blk, D_IN), lambda i: (i, 0)),
            pl.BlockSpec((16, D_IN), lambda i: (0, 0)),
        ],
        out_specs=pl.BlockSpec((blk, D_IN), lambda i: (i, 0)),
        out_shape=jax.ShapeDtypeStruct((N_V, D_IN), jnp.float32),
    )(agg2, xres, expand)


def _gather_scatter(xt, aecat, consts):
    bd, sel, expand, zeros, vg, es, eg, vs = consts
    agg_e = _seg_sum(xt, vg, es, zeros, NE_ACC, SPLIT_A)
    agg_e = agg_e.reshape(NC, NE_ACC, WID)[:, :N_E, :]
    z = _edge_stage(agg_e, aecat, bd, sel)
    agg_v = _seg_sum(z, eg, vs, zeros, NV_ACC, SPLIT_B)
    return agg_v.reshape(NC, NV_ACC, WID)[:, :N_V, :]


def kernel(x, v_idx, e_idx, W1, b1, ae1, W2, b2, ae2):
    f32 = jnp.float32
    x1, x2 = x[:, :D_IN], x[:, D_IN:]
    w1c = jnp.transpose(W1, (1, 0, 2)).reshape(D_IN, D_IN)
    w2c = jnp.transpose(W2, (1, 0, 2)).reshape(D_IN, D_IN)
    b1c, b2c = b1.reshape(D_IN), b2.reshape(D_IN)
    ae1c, ae2c = ae1.reshape(D_IN), ae2.reshape(D_IN)

    heads = jnp.arange(D_IN, dtype=jnp.int32) // D_HEAD
    bd = (heads[:, None] == heads[None, :]).astype(f32)
    col = jnp.arange(16, dtype=jnp.int32)
    sel = ((jnp.arange(D_IN)[:, None] == col[None, :] * D_HEAD)
           & (col[None, :] < H)).astype(f32)
    expand = (col[:, None] == heads[None, :]).astype(f32)

    zeros = jnp.zeros((NV_ACC // NS, WID), f32)
    npad = P_PAD - N_PAIRS
    v32 = v_idx.astype(jnp.int32)
    e32 = e_idx.astype(jnp.int32)
    pad0 = jnp.zeros((npad,), jnp.int32)
    vg = jnp.concatenate([v32, pad0])
    eg = jnp.concatenate([e32, pad0])
    es = jnp.concatenate([e32, jnp.full((npad,), N_E, jnp.int32)])
    vs = jnp.concatenate([v32, jnp.full((npad,), N_V, jnp.int32)])

    consts = (bd, sel, expand, zeros, vg, es, eg, vs)
    xt1 = _theta(x2, w1c, b1c)
    agg_v1 = _gather_scatter(xt1, ae1c, consts)
    y1, xt2 = _vertex_theta(agg_v1, x1, expand, w2c, b2c)
    agg_v2 = _gather_scatter(xt2, ae2c, consts)
    return _vertex_cat(agg_v2, x2, expand, y1)


# speedup vs baseline: 1.3288x; 1.0332x over previous
import functools

import jax
import jax.numpy as jnp
from jax import lax
from jax.experimental import pallas as pl
from jax.experimental.pallas import tpu as pltpu
from jax.experimental.pallas import tpu_sc as plsc

N_V = 10000
N_E = 5000
N_PAIRS = 160000
C = 256
H = 4
D_IN = C // 2
D_HEAD = D_IN // H
WID = 144

NC = 2
NS = 16
P_PAD = 163840
SPLIT_A = (64, 16)
SPLIT_B = (124, 36)

NE_ACC = 5120
NV_ACC = 10016


def _seg_sum_body(nacc, ch, n0, n1, data, gidx, sidx, zeros, out, acc, gi_v,
                  si_v, rows_v, sem):
    c = lax.axis_index("c")
    s = lax.axis_index("s")
    rpt = nacc // NS

    pltpu.sync_copy(zeros.at[pl.ds(0, rpt)], acc.at[pl.ds(s * rpt, rpt)])
    plsc.subcore_barrier()

    def run(base, n):
        pltpu.sync_copy(gidx.at[pl.ds(base, n)], gi_v.at[pl.ds(0, n)])
        pltpu.sync_copy(sidx.at[pl.ds(base, n)], si_v.at[pl.ds(0, n)])

        pltpu.async_copy(data.at[gi_v.at[0]], rows_v.at[0], sem)

        def chunk(j, carry):
            p = lax.rem(j, 2)
            pltpu.make_async_copy(data.at[gi_v.at[j]], rows_v.at[p],
                                  sem).wait()

            @pl.when(j + 1 < n)
            def _():
                pltpu.async_copy(data.at[gi_v.at[j + 1]], rows_v.at[1 - p],
                                 sem)

            pltpu.sync_copy(rows_v.at[p], acc.at[si_v.at[j]], add=True)
            return carry

        lax.fori_loop(0, n, chunk, 0)

    @pl.when(c == 0)
    def _():
        run(s * n0, n0)

    @pl.when(c == 1)
    def _():
        run(NS * n0 + s * n1, n1)

    plsc.subcore_barrier()

    r0 = s * rpt
    pltpu.sync_copy(acc.at[pl.ds(r0, rpt)],
                    out.at[pl.ds(c * nacc + r0, rpt)])


def _seg_sum(data, gidx, sidx, zeros, nacc, split):
    n0, n1 = split
    ch = P_PAD // ((n0 + n1) * NS)
    body = functools.partial(_seg_sum_body, nacc, ch, n0, n1)
    f = pl.kernel(
        body,
        out_type=jax.ShapeDtypeStruct((NC * nacc, WID), jnp.float32),
        mesh=plsc.VectorSubcoreMesh(core_axis_name="c", subcore_axis_name="s"),
        scratch_types=[
            pltpu.VMEM_SHARED((nacc, WID), jnp.float32),
            pltpu.VMEM((max(n0, n1), ch), jnp.int32),
            pltpu.VMEM((max(n0, n1), ch), jnp.int32),
            pltpu.VMEM((2, ch, WID), jnp.float32),
            pltpu.SemaphoreType.DMA,
        ],
        compiler_params=pltpu.CompilerParams(use_tc_tiling_on_sc=False),
    )
    return f(data, gidx.reshape(-1, ch), sidx.reshape(-1, ch), zeros)


def _theta_body(x_ref, w_ref, b_ref, o_ref):
    xt = jnp.dot(x_ref[...], w_ref[...],
                 preferred_element_type=jnp.float32) + b_ref[...]
    extra = jnp.broadcast_to(
        (lax.broadcasted_iota(jnp.int32, (1, 16), 1) == 0)
        .astype(jnp.float32), (xt.shape[0], 16))
    o_ref[...] = jnp.concatenate([xt, extra], axis=1)


def _theta(x, wcat, bcat):
    n = x.shape[0]
    blk = 2000
    return pl.pallas_call(
        _theta_body,
        grid=(n // blk,),
        in_specs=[
            pl.BlockSpec((blk, D_IN), lambda i: (i, 0)),
            pl.BlockSpec((D_IN, D_IN), lambda i: (0, 0)),
            pl.BlockSpec((1, D_IN), lambda i: (0, 0)),
        ],
        out_specs=pl.BlockSpec((blk, WID), lambda i: (i, 0)),
        out_shape=jax.ShapeDtypeStruct((n, WID), jnp.float32),
    )(x, wcat, bcat.reshape(1, D_IN))


def _edge_stage_body(agg_ref, ae_ref, bd_ref, sel_ref, z_ref):
    s = agg_ref[0] + agg_ref[1]
    deg = jnp.maximum(s[:, D_IN:D_IN + 1], 1.0)
    y = s[:, :D_IN] / deg
    p = y * ae_ref[...]
    alpha = jnp.dot(p, bd_ref[...],
                    preferred_element_type=jnp.float32)
    alpha = jnp.where(alpha > 0, alpha, 0.2 * alpha)
    m = jnp.max(alpha, axis=0, keepdims=True)
    e = jnp.exp(alpha - m)
    extra = jnp.dot(e, sel_ref[...],
                    preferred_element_type=jnp.float32)
    z_ref[...] = jnp.concatenate([y * e, extra], axis=1)


def _edge_stage(agg2, aecat, bd, sel):
    return pl.pallas_call(
        _edge_stage_body,
        in_specs=[
            pl.BlockSpec((2, N_E, WID), lambda: (0, 0, 0)),
            pl.BlockSpec((1, D_IN), lambda: (0, 0)),
            pl.BlockSpec((D_IN, D_IN), lambda: (0, 0)),
            pl.BlockSpec((D_IN, 16), lambda: (0, 0)),
        ],
        out_specs=pl.BlockSpec((N_E, WID), lambda: (0, 0)),
        out_shape=jax.ShapeDtypeStruct((N_E, WID), jnp.float32),
    )(agg2, aecat.reshape(1, D_IN), bd, sel)


def _vertex_theta_body(agg_ref, xres_ref, exp_ref, w_ref, b_ref, y_ref,
                       xt_ref):
    s = agg_ref[0] + agg_ref[1]
    den = jnp.dot(s[:, D_IN:], exp_ref[...],
                  preferred_element_type=jnp.float32)
    out = s[:, :D_IN] / jnp.maximum(den, 1e-12)
    out = jnp.where(out > 0, out, jnp.exp(out) - 1.0)
    y = xres_ref[...] + out
    y_ref[...] = y
    xt = jnp.dot(y, w_ref[...],
                 preferred_element_type=jnp.float32) + b_ref[...]
    extra = jnp.broadcast_to(
        (lax.broadcasted_iota(jnp.int32, (1, 16), 1) == 0)
        .astype(jnp.float32), (xt.shape[0], 16))
    xt_ref[...] = jnp.concatenate([xt, extra], axis=1)


def _vertex_theta(agg2, xres, expand, wcat, bcat):
    blk = 2000
    return pl.pallas_call(
        _vertex_theta_body,
        grid=(N_V // blk,),
        in_specs=[
            pl.BlockSpec((2, blk, WID), lambda i: (0, i, 0)),
            pl.BlockSpec((blk, D_IN), lambda i: (i, 0)),
            pl.BlockSpec((16, D_IN), lambda i: (0, 0)),
            pl.BlockSpec((D_IN, D_IN), lambda i: (0, 0)),
            pl.BlockSpec((1, D_IN), lambda i: (0, 0)),
        ],
        out_specs=[
            pl.BlockSpec((blk, D_IN), lambda i: (i, 0)),
            pl.BlockSpec((blk, WID), lambda i: (i, 0)),
        ],
        out_shape=[
            jax.ShapeDtypeStruct((N_V, D_IN), jnp.float32),
            jax.ShapeDtypeStruct((N_V, WID), jnp.float32),
        ],
    )(agg2, xres, expand, wcat, bcat.reshape(1, D_IN))


def _vertex_cat_body(agg_ref, xres_ref, exp_ref, y1_ref, o_ref):
    s = agg_ref[0] + agg_ref[1]
    den = jnp.dot(s[:, D_IN:], exp_ref[...],
                  preferred_element_type=jnp.float32)
    out = s[:, :D_IN] / jnp.maximum(den, 1e-12)
    out = jnp.where(out > 0, out, jnp.exp(out) - 1.0)
    o_ref[...] = jnp.concatenate([y1_ref[...], xres_ref[...] + out], axis=1)


def _vertex_cat(agg2, xres, expand, y1):
    blk = 2000
    return pl.pallas_call(
        _vertex_cat_body,
        grid=(N_V // blk,),
        in_specs=[
            pl.BlockSpec((2, blk, WID), lambda i: (0, i, 0)),
            pl.BlockSpec((blk, D_IN), lambda i: (i, 0)),
            pl.BlockSpec((16, D_IN), lambda i: (0, 0)),
            pl.BlockSpec((blk, D_IN), lambda i: (i, 0)),
        ],
        out_specs=pl.BlockSpec((blk, C), lambda i: (i, 0)),
        out_shape=jax.ShapeDtypeStruct((N_V, C), jnp.float32),
    )(agg2, xres, expand, y1)


def _vertex_stage_body(agg_ref, xres_ref, exp_ref, y_ref):
    s = agg_ref[0] + agg_ref[1]
    den = jnp.dot(s[:, D_IN:], exp_ref[...],
                  preferred_element_type=jnp.float32)
    out = s[:, :D_IN] / jnp.maximum(den, 1e-12)
    out = jnp.where(out > 0, out, jnp.exp(out) - 1.0)
    y_ref[...] = xres_ref[...] + out


def _vertex_stage(agg2, xres, expand):
    blk = 2000
    return pl.pallas_call(
        _vertex_stage_body,
        grid=(N_V // blk,),
        in_specs=[
            pl.BlockSpec((2, blk, WID), lambda i: (0, i, 0)),
            pl.BlockSpec((blk, D_IN), lambda i: (i, 0)),
            pl.BlockSpec((16, D_IN), lambda i: (0, 0)),
        ],
        out_specs=pl.BlockSpec((blk, D_IN), lambda i: (i, 0)),
        out_shape=jax.ShapeDtypeStruct((N_V, D_IN), jnp.float32),
    )(agg2, xres, expand)


def _gather_scatter(xt, aecat, consts):
    bd, sel, expand, zeros, vg, es, eg, vs = consts
    agg_e = _seg_sum(xt, vg, es, zeros, NE_ACC, SPLIT_A)
    agg_e = agg_e.reshape(NC, NE_ACC, WID)[:, :N_E, :]
    z = _edge_stage(agg_e, aecat, bd, sel)
    agg_v = _seg_sum(z, eg, vs, zeros, NV_ACC, SPLIT_B)
    return agg_v.reshape(NC, NV_ACC, WID)[:, :N_V, :]


def kernel(x, v_idx, e_idx, W1, b1, ae1, W2, b2, ae2):
    f32 = jnp.float32
    x1, x2 = x[:, :D_IN], x[:, D_IN:]
    w1c = jnp.transpose(W1, (1, 0, 2)).reshape(D_IN, D_IN)
    w2c = jnp.transpose(W2, (1, 0, 2)).reshape(D_IN, D_IN)
    b1c, b2c = b1.reshape(D_IN), b2.reshape(D_IN)
    ae1c, ae2c = ae1.reshape(D_IN), ae2.reshape(D_IN)

    heads = jnp.arange(D_IN, dtype=jnp.int32) // D_HEAD
    bd = (heads[:, None] == heads[None, :]).astype(f32)
    col = jnp.arange(16, dtype=jnp.int32)
    sel = ((jnp.arange(D_IN)[:, None] == col[None, :] * D_HEAD)
           & (col[None, :] < H)).astype(f32)
    expand = (col[:, None] == heads[None, :]).astype(f32)

    zeros = jnp.zeros((NV_ACC // NS, WID), f32)
    npad = P_PAD - N_PAIRS
    v32 = v_idx.astype(jnp.int32)
    e32 = e_idx.astype(jnp.int32)
    pad0 = jnp.zeros((npad,), jnp.int32)
    vg = jnp.concatenate([v32, pad0])
    eg = jnp.concatenate([e32, pad0])
    es = jnp.concatenate([e32, jnp.full((npad,), N_E, jnp.int32)])
    vs = jnp.concatenate([v32, jnp.full((npad,), N_V, jnp.int32)])

    consts = (bd, sel, expand, zeros, vg, es, eg, vs)
    xt1 = _theta(x2, w1c, b1c)
    agg_v1 = _gather_scatter(xt1, ae1c, consts)
    y1, xt2 = _vertex_theta(agg_v1, x1, expand, w2c, b2c)
    agg_v2 = _gather_scatter(xt2, ae2c, consts)
    return _vertex_cat(agg_v2, x2, expand, y1)
